# baseline (speedup 1.0000x reference)
import functools

import jax
import jax.numpy as jnp
from jax import lax
from jax.experimental import pallas as pl
from jax.experimental.pallas import tpu as pltpu
from jax.experimental.pallas import tpu_sc as plsc

N_EXP = 64
N_GRP = 8
EPG = 8
TOPK_G = 3
TOPK = 8
LANES = 16
INT_MIN = -2147483648


def _logits_kernel(w_ref, h_ref, out_ref):
    out_ref[...] = jax.lax.dot_general(
        w_ref[...], h_ref[...],
        (((1,), (1,)), ((), ())),
        preferred_element_type=jnp.float32,
    )


def _tree_reduce(vals, op):
    while len(vals) > 1:
        nxt = [op(vals[i], vals[i + 1]) for i in range(0, len(vals) - 1, 2)]
        if len(vals) % 2:
            nxt.append(vals[-1])
        vals = nxt
    return vals[0]


def _sortable(v):
    b = plsc.bitcast(v, jnp.int32)
    return jnp.where(b < 0, b ^ 0x7FFFFFFF, b)


def _make_sc_router(S):
    info = plsc.get_sparse_core_info()
    nc, ns = info.num_cores, info.num_subcores
    nw = nc * ns
    tpw = S // nw
    nchunk = tpw // LANES
    mesh = plsc.VectorSubcoreMesh(core_axis_name="c", subcore_axis_name="s")

    @functools.partial(
        pl.kernel,
        out_type=jax.ShapeDtypeStruct((S, TOPK), jnp.float32),
        mesh=mesh,
        scratch_types=[
            pltpu.VMEM((N_EXP, tpw), jnp.float32),
            pltpu.VMEM((tpw, TOPK), jnp.float32),
        ],
        compiler_params=pltpu.CompilerParams(needs_layout_passes=False),
    )
    def router(lt_hbm, out_hbm, lt_v, out_v):
        wid = lax.axis_index("s") * nc + lax.axis_index("c")
        base = wid * tpw
        pltpu.sync_copy(lt_hbm.at[:, pl.ds(base, tpw)], lt_v)
        iota = lax.iota(jnp.int32, LANES)

        def chunk_body(c, carry):
            off = c * LANES
            pos = off + iota

            gk = []
            for g in range(N_GRP):
                rows = [lt_v[g * EPG + j, pl.ds(off, LANES)]
                        for j in range(EPG)]
                gm = _tree_reduce(rows, jnp.maximum)
                gk.append((_sortable(gm) & -8) | (7 - g))

            sel_g = []
            for _ in range(TOPK_G):
                m = _tree_reduce(list(gk), jnp.maximum)
                g_r = 7 - (m & 7)
                sel_g.append(g_r)
                gk = [jnp.where(g_r == g, INT_MIN, gk[g])
                      for g in range(N_GRP)]

            ck = []
            cidx = []
            for r in range(TOPK_G):
                ebase = sel_g[r] * EPG
                for j in range(EPG):
                    ei = ebase + j
                    cv = plsc.load_gather(lt_v, [ei, pos])
                    cidx.append(ei)
                    ck.append((_sortable(cv) & -64) | (63 - ei))

            recs = []
            for _ in range(TOPK):
                m = _tree_reduce(list(ck), jnp.maximum)
                recs.append(m)
                ck = [jnp.where(k == m, INT_MIN, k) for k in ck]

            vals = []
            for i in range(TOPK):
                ei = 63 - (recs[i] & 63)
                vals.append(plsc.load_gather(lt_v, [ei, pos]))
            ex = [jnp.exp(v - vals[0]) for v in vals]
            s = _tree_reduce(list(ex), jnp.add)
            rcp = 1.0 / s
            for i in range(TOPK):
                plsc.store_scatter(
                    out_v, [pos, jnp.full((LANES,), i, jnp.int32)],
                    ex[i] * rcp)
            return carry

        lax.fori_loop(0, nchunk, chunk_body, 0)
        pltpu.sync_copy(out_v, out_hbm.at[pl.ds(base, tpw), :])

    return router


def kernel(hidden_states, kernel):
    gate_w = kernel
    S, H = hidden_states.shape
    T = 1024
    P = 4
    SC = S // P
    mm = pl.pallas_call(
        _logits_kernel,
        grid=(SC // T,),
        in_specs=[
            pl.BlockSpec((N_EXP, H), lambda i: (0, 0)),
            pl.BlockSpec((T, H), lambda i: (i, 0)),
        ],
        out_specs=pl.BlockSpec((N_EXP, T), lambda i: (0, i)),
        out_shape=jax.ShapeDtypeStruct((N_EXP, SC), jnp.float32),
    )
    router = _make_sc_router(SC)
    outs = []
    for p in range(P):
        logits_t = mm(gate_w, jax.lax.slice(hidden_states, (p * SC, 0), ((p + 1) * SC, H)))
        outs.append(router(logits_t))
    return jnp.concatenate(outs, axis=0)


# speedup vs baseline: 2.0206x; 2.0206x over previous
import functools

import jax
import jax.numpy as jnp
from jax import lax
from jax.experimental import pallas as pl
from jax.experimental.pallas import tpu as pltpu
from jax.experimental.pallas import tpu_sc as plsc

N_EXP = 64
N_GRP = 8
EPG = 8
TOPK_G = 3
TOPK = 8
LANES = 16
INT_MIN = -2147483648


def _logits_kernel(w_ref, h_ref, out_ref):
    out_ref[...] = jax.lax.dot_general(
        w_ref[...], h_ref[...],
        (((1,), (1,)), ((), ())),
        preferred_element_type=jnp.float32,
    )


def _tree_reduce(vals, op):
    while len(vals) > 1:
        nxt = [op(vals[i], vals[i + 1]) for i in range(0, len(vals) - 1, 2)]
        if len(vals) % 2:
            nxt.append(vals[-1])
        vals = nxt
    return vals[0]


def _sortable(v):
    b = plsc.bitcast(v, jnp.int32)
    return jnp.where(b < 0, b ^ 0x7FFFFFFF, b)


def _make_sc_router(S):
    info = plsc.get_sparse_core_info()
    nc, ns = info.num_cores, info.num_subcores
    nw = nc * ns
    tpw = S // nw
    nchunk = tpw // LANES
    mesh = plsc.VectorSubcoreMesh(core_axis_name="c", subcore_axis_name="s")

    @functools.partial(
        pl.kernel,
        out_type=jax.ShapeDtypeStruct((S, TOPK), jnp.float32),
        mesh=mesh,
        scratch_types=[
            pltpu.VMEM((N_EXP, tpw), jnp.float32),
            pltpu.VMEM((tpw, TOPK), jnp.float32),
        ],
        compiler_params=pltpu.CompilerParams(needs_layout_passes=False),
    )
    def router(lt_hbm, out_hbm, lt_v, out_v):
        wid = lax.axis_index("s") * nc + lax.axis_index("c")
        base = wid * tpw
        pltpu.sync_copy(lt_hbm.at[:, pl.ds(base, tpw)], lt_v)
        iota = lax.iota(jnp.int32, LANES)

        def chunk_body(c, carry):
            off = c * LANES
            pos = off + iota

            gk = []
            for g in range(N_GRP):
                rows = [lt_v[g * EPG + j, pl.ds(off, LANES)]
                        for j in range(EPG)]
                gm = _tree_reduce(rows, jnp.maximum)
                gk.append((_sortable(gm) & -8) | (7 - g))

            sel_g = []
            for _ in range(TOPK_G):
                m = _tree_reduce(list(gk), jnp.maximum)
                g_r = 7 - (m & 7)
                sel_g.append(g_r)
                gk = [jnp.where(g_r == g, INT_MIN, gk[g])
                      for g in range(N_GRP)]

            ck = []
            cidx = []
            for r in range(TOPK_G):
                ebase = sel_g[r] * EPG
                for j in range(EPG):
                    ei = ebase + j
                    cv = plsc.load_gather(lt_v, [ei, pos])
                    cidx.append(ei)
                    ck.append((_sortable(cv) & -64) | (63 - ei))

            recs = []
            for _ in range(TOPK):
                m = _tree_reduce(list(ck), jnp.maximum)
                recs.append(m)
                ck = [jnp.where(k == m, INT_MIN, k) for k in ck]

            vals = []
            for i in range(TOPK):
                ei = 63 - (recs[i] & 63)
                vals.append(plsc.load_gather(lt_v, [ei, pos]))
            ex = [jnp.exp(v - vals[0]) for v in vals]
            s = _tree_reduce(list(ex), jnp.add)
            rcp = 1.0 / s
            for i in range(TOPK):
                plsc.store_scatter(
                    out_v, [pos, jnp.full((LANES,), i, jnp.int32)],
                    ex[i] * rcp)
            return carry

        lax.fori_loop(0, nchunk, chunk_body, 0)
        pltpu.sync_copy(out_v, out_hbm.at[pl.ds(base, tpw), :])

    return router


def kernel(hidden_states, kernel):
    gate_w = kernel
    S, H = hidden_states.shape
    T = 1024
    P = 4
    SC = S // P
    router = _make_sc_router(SC)
    outs = []
    for p in range(P):
        off = p * (SC // T)
        mm = pl.pallas_call(
            _logits_kernel,
            grid=(SC // T,),
            in_specs=[
                pl.BlockSpec((N_EXP, H), lambda i: (0, 0)),
                pl.BlockSpec((T, H), lambda i, o=off: (i + o, 0)),
            ],
            out_specs=pl.BlockSpec((N_EXP, T), lambda i: (0, i)),
            out_shape=jax.ShapeDtypeStruct((N_EXP, SC), jnp.float32),
        )
        outs.append(router(mm(gate_w, hidden_states)))
    return jnp.concatenate(outs, axis=0)


# speedup vs baseline: 2.8462x; 1.4086x over previous
import functools

import jax
import jax.numpy as jnp
from jax.experimental import pallas as pl
from jax.experimental.pallas import tpu as pltpu

N_EXP = 64
N_GRP = 8
EPG = 8
TOPK_G = 3
TOPK = 8
NEG = -1e30


def _gate_kernel(w_ref, h_ref, out_ref):
    logits = jax.lax.dot_general(
        w_ref[...], h_ref[...],
        (((1,), (1,)), ((), ())),
        preferred_element_type=jnp.float32,
    )
    T = logits.shape[1]

    gs = jnp.concatenate(
        [jnp.max(logits[g * EPG:(g + 1) * EPG], axis=0, keepdims=True)
         for g in range(N_GRP)], axis=0)

    gidx = jax.lax.broadcasted_iota(jnp.int32, (N_GRP, T), 0)
    cur = gs
    sel = jnp.zeros((N_GRP, T), jnp.bool_)
    for _ in range(TOPK_G):
        m = jnp.max(cur, axis=0, keepdims=True)
        cand = jnp.where(cur == m, gidx, N_GRP)
        amin = jnp.min(cand, axis=0, keepdims=True)
        pick = gidx == amin
        sel = jnp.logical_or(sel, pick)
        cur = jnp.where(pick, NEG, cur)

    sel64 = jnp.concatenate(
        [jnp.broadcast_to(sel[g:g + 1], (EPG, T)) for g in range(N_GRP)],
        axis=0)
    masked = jnp.where(sel64, logits, NEG)

    eidx = jax.lax.broadcasted_iota(jnp.int32, (N_EXP, T), 0)
    vals = []
    for _ in range(TOPK):
        m = jnp.max(masked, axis=0, keepdims=True)
        vals.append(m)
        cand = jnp.where(masked == m, eidx, N_EXP)
        amin = jnp.min(cand, axis=0, keepdims=True)
        masked = jnp.where(eidx == amin, NEG, masked)

    w = jnp.concatenate(vals, axis=0)
    e = jnp.exp(w - w[0:1])
    out = e / jnp.sum(e, axis=0, keepdims=True)
    out_ref[...] = out.T


def kernel(hidden_states, kernel):
    gate_w = kernel
    S, H = hidden_states.shape
    T = 1024
    return pl.pallas_call(
        _gate_kernel,
        grid=(S // T,),
        in_specs=[
            pl.BlockSpec((N_EXP, H), lambda i: (0, 0)),
            pl.BlockSpec((T, H), lambda i: (i, 0)),
        ],
        out_specs=pl.BlockSpec((T, TOPK), lambda i: (i, 0)),
        out_shape=jax.ShapeDtypeStruct((S, TOPK), jnp.float32),
    )(gate_w, hidden_states)


# speedup vs baseline: 3.0428x; 1.0691x over previous
import functools

import jax
import jax.numpy as jnp
from jax.experimental import pallas as pl
from jax.experimental.pallas import tpu as pltpu

N_EXP = 64
N_GRP = 8
EPG = 8
TOPK_G = 3
TOPK = 8
NEG = -1e30


def _gate_kernel(w_ref, h_ref, out_ref):
    logits = jax.lax.dot_general(
        w_ref[...], h_ref[...],
        (((1,), (1,)), ((), ())),
        preferred_element_type=jnp.float32,
    )
    T = logits.shape[1]

    gs = jnp.concatenate(
        [jnp.max(logits[g * EPG:(g + 1) * EPG], axis=0, keepdims=True)
         for g in range(N_GRP)], axis=0)

    gidx = jax.lax.broadcasted_iota(jnp.int32, (N_GRP, T), 0)
    cur = gs
    sel = jnp.zeros((N_GRP, T), jnp.bool_)
    for _ in range(TOPK_G):
        m = jnp.max(cur, axis=0, keepdims=True)
        cand = jnp.where(cur == m, gidx, N_GRP)
        amin = jnp.min(cand, axis=0, keepdims=True)
        pick = gidx == amin
        sel = jnp.logical_or(sel, pick)
        cur = jnp.where(pick, NEG, cur)

    sel64 = jnp.concatenate(
        [jnp.broadcast_to(sel[g:g + 1], (EPG, T)) for g in range(N_GRP)],
        axis=0)
    masked = jnp.where(sel64, logits, NEG)

    eidx = jax.lax.broadcasted_iota(jnp.int32, (N_EXP, T), 0)
    vals = []
    for _ in range(TOPK):
        m = jnp.max(masked, axis=0, keepdims=True)
        vals.append(m)
        cand = jnp.where(masked == m, eidx, N_EXP)
        amin = jnp.min(cand, axis=0, keepdims=True)
        masked = jnp.where(eidx == amin, NEG, masked)

    w = jnp.concatenate(vals, axis=0)
    e = jnp.exp(w - w[0:1])
    out = e / jnp.sum(e, axis=0, keepdims=True)
    out_ref[...] = out.T


def kernel(hidden_states, kernel):
    gate_w = kernel
    S, H = hidden_states.shape
    T = 2048
    return pl.pallas_call(
        _gate_kernel,
        grid=(S // T,),
        in_specs=[
            pl.BlockSpec((N_EXP, H), lambda i: (0, 0)),
            pl.BlockSpec((T, H), lambda i: (i, 0)),
        ],
        out_specs=pl.BlockSpec((T, TOPK), lambda i: (i, 0)),
        out_shape=jax.ShapeDtypeStruct((S, TOPK), jnp.float32),
    )(gate_w, hidden_states)
